# Initial kernel scaffold; baseline (speedup 1.0000x reference)
import jax
import jax.numpy as jnp
from jax.experimental import pallas as pl


def kernel(src, flow):
    raise NotImplementedError("write your pallas kernel here")


# speedup vs baseline: 2.4217x; 2.4217x over previous
import functools

import jax
import jax.numpy as jnp
from jax import lax
from jax.experimental import pallas as pl
from jax.experimental.pallas import tpu as pltpu
from jax.experimental.pallas import tpu_sc as plsc

_B, _C, _H, _W = 8, 16, 512, 512
_HW = _H * _W
_NPIX = _B * _HW
_NW = 32
_PIX_PER_W = _NPIX // _NW
_CH = 512
_NCHUNK = _PIX_PER_W // _CH
_L = 16
_G = _CH // _L
_ISL = 128
_NSL = 4 * _CH // _ISL


def _body(table, fx_hbm, fy_hbm, out_hbm, fx_v, fy_v, idx_v, w_v, rows_v,
          out_v, sem):
    cid = lax.axis_index("c")
    sid = lax.axis_index("s")
    wid = cid * 16 + sid
    lane = jnp.arange(_L, dtype=jnp.int32)

    def chunk_body(t, carry):
        base_pix = wid * _PIX_PER_W + t * _CH
        pltpu.sync_copy(fx_hbm.at[pl.ds(base_pix, _CH)], fx_v)
        pltpu.sync_copy(fy_hbm.at[pl.ds(base_pix, _CH)], fy_v)

        def gen_body(g, carry2):
            p = base_pix + g * _L + lane
            q = p & (_HW - 1)
            hh = q >> 9
            ww = q & (_W - 1)
            row0 = p - q

            fx = fx_v[pl.ds(g * _L, _L)]
            fy = fy_v[pl.ds(g * _L, _L)]
            x = ww.astype(jnp.float32) + fx
            y = hh.astype(jnp.float32) + fy
            x = jnp.minimum(jnp.maximum(x, -4.0), float(_W) + 4.0)
            y = jnp.minimum(jnp.maximum(y, -4.0), float(_H) + 4.0)
            xt = x.astype(jnp.int32)
            x0 = jnp.where(xt.astype(jnp.float32) > x, xt - 1, xt)
            yt = y.astype(jnp.int32)
            y0 = jnp.where(yt.astype(jnp.float32) > y, yt - 1, yt)
            dx = x - x0.astype(jnp.float32)
            dy = y - y0.astype(jnp.float32)
            one = jnp.float32(1.0)
            zero = jnp.float32(0.0)
            vx0 = jnp.where((x0 >= 0) & (x0 <= _W - 1), one, zero)
            vx1 = jnp.where((x0 >= -1) & (x0 <= _W - 2), one, zero)
            vy0 = jnp.where((y0 >= 0) & (y0 <= _H - 1), one, zero)
            vy1 = jnp.where((y0 >= -1) & (y0 <= _H - 2), one, zero)
            cx0 = jnp.minimum(jnp.maximum(x0, 0), _W - 1)
            cx1 = jnp.minimum(jnp.maximum(x0 + 1, 0), _W - 1)
            cy0 = jnp.minimum(jnp.maximum(y0, 0), _H - 1) << 9
            cy1 = jnp.minimum(jnp.maximum(y0 + 1, 0), _H - 1) << 9

            col = (g & 7) * _L
            r = g >> 3
            idx_v[r, pl.ds(col, _L)] = row0 + cy0 + cx0
            idx_v[r + 4, pl.ds(col, _L)] = row0 + cy0 + cx1
            idx_v[r + 8, pl.ds(col, _L)] = row0 + cy1 + cx0
            idx_v[r + 12, pl.ds(col, _L)] = row0 + cy1 + cx1

            omdx = one - dx
            omdy = one - dy
            s = pl.ds(g * _L, _L)
            w_v[0, s] = omdx * omdy * (vx0 * vy0)
            w_v[1, s] = dx * omdy * (vx1 * vy0)
            w_v[2, s] = omdx * dy * (vx0 * vy1)
            w_v[3, s] = dx * dy * (vx1 * vy1)
            return carry2

        lax.fori_loop(0, _G, gen_body, 0)

        copies = []
        for k in range(_NSL):
            copies.append(pltpu.async_copy(
                table.at[idx_v.at[k]],
                rows_v.at[pl.ds(k * _ISL, _ISL)],
                sem,
            ))
        for c in copies:
            c.wait()

        def blend_body(g, carry2):
            s = pl.ds(g * _L, _L)
            wa = w_v[0, s]
            wb = w_v[1, s]
            wc = w_v[2, s]
            wd = w_v[3, s]
            zero = jnp.float32(0.0)
            for j in range(_L):
                onehot = lane == j
                was = jnp.sum(jnp.where(onehot, wa, zero))
                wbs = jnp.sum(jnp.where(onehot, wb, zero))
                wcs = jnp.sum(jnp.where(onehot, wc, zero))
                wds = jnp.sum(jnp.where(onehot, wd, zero))
                pp = g * _L + j
                ra = rows_v[pp, :]
                rb = rows_v[pp + _CH, :]
                rc = rows_v[pp + 2 * _CH, :]
                rd = rows_v[pp + 3 * _CH, :]
                out_v[pp, :] = was * ra + wbs * rb + wcs * rc + wds * rd
            return carry2

        lax.fori_loop(0, _G, blend_body, 0)

        pltpu.sync_copy(out_v, out_hbm.at[pl.ds(base_pix, _CH)])
        return carry

    lax.fori_loop(0, _NCHUNK, chunk_body, 0)


_warp_sc = pl.kernel(
    _body,
    out_type=jax.ShapeDtypeStruct((_NPIX, _C), jnp.float32),
    mesh=plsc.VectorSubcoreMesh(core_axis_name="c", subcore_axis_name="s"),
    compiler_params=pltpu.CompilerParams(
        needs_layout_passes=False, use_tc_tiling_on_sc=False
    ),
    scratch_types=[
        pltpu.VMEM((_CH,), jnp.float32),
        pltpu.VMEM((_CH,), jnp.float32),
        pltpu.VMEM((_NSL, _ISL), jnp.int32),
        pltpu.VMEM((4, _CH), jnp.float32),
        pltpu.VMEM((4 * _CH, _C), jnp.float32),
        pltpu.VMEM((_CH, _C), jnp.float32),
        pltpu.SemaphoreType.DMA,
    ],
)


def kernel(src, flow):
    table = jnp.transpose(src, (0, 2, 3, 1)).reshape(_NPIX, _C)
    fx = flow[:, 0, :, :].reshape(_NPIX)
    fy = flow[:, 1, :, :].reshape(_NPIX)
    out_cm = _warp_sc(table, fx, fy)
    return jnp.transpose(out_cm.reshape(_B, _H, _W, _C), (0, 3, 1, 2))
